# baseline (speedup 1.0000x reference)
import functools

import jax
import jax.numpy as jnp
from jax import lax
from jax.experimental import pallas as pl
from jax.experimental.pallas import tpu as pltpu
from jax.experimental.pallas import tpu_sc as plsc

_NC = 2
_NS = 16
_NW = _NC * _NS

_CH = 128
_K = 8


@functools.partial(jax.jit, static_argnums=(2, 3, 4))
def _emb_call(x_blocks, table, groups, ch, k):
    d = table.shape[1]
    mesh = plsc.VectorSubcoreMesh(core_axis_name="c", subcore_axis_name="s")

    @functools.partial(
        pl.kernel,
        mesh=mesh,
        out_type=jax.ShapeDtypeStruct((_NW * groups, k, ch, d), jnp.float32),
        scratch_types=[
            pltpu.VMEM((k, ch), jnp.int32),
            pltpu.VMEM((k, ch, d), jnp.float32),
            pltpu.SemaphoreType.DMA,
        ],
        compiler_params=pltpu.CompilerParams(use_tc_tiling_on_sc=False),
    )
    def emb(x_hbm, table_hbm, out_hbm, idx_v, rows_v, sem):
        wid = lax.axis_index("s") * _NC + lax.axis_index("c")

        def body(g, carry):
            blk = wid * groups + g
            pltpu.sync_copy(x_hbm.at[blk], idx_v)
            copies = [
                pltpu.async_copy(table_hbm.at[idx_v.at[b]], rows_v.at[b], sem)
                for b in range(k)
            ]
            for cp in copies:
                cp.wait()
            pltpu.sync_copy(rows_v, out_hbm.at[blk])
            return carry

        lax.fori_loop(0, groups, body, 0)

    return emb(x_blocks, table)


def kernel(x, table):
    b, l = x.shape
    _, d = table.shape
    n = b * l
    rows_per_group = _K * _CH
    groups = n // (_NW * rows_per_group)
    x_blocks = x.reshape(_NW * groups, _K, _CH).astype(jnp.int32)
    out = _emb_call(x_blocks, table, groups, _CH, _K)
    return out.reshape(b, l, d)


# speedup vs baseline: 1.0158x; 1.0158x over previous
import functools

import jax
import jax.numpy as jnp
from jax import lax
from jax.experimental import pallas as pl
from jax.experimental.pallas import tpu as pltpu
from jax.experimental.pallas import tpu_sc as plsc

_NC = 2
_NS = 16
_NW = _NC * _NS

_CH = 128
_K = 2
_NBUF = 4


@functools.partial(jax.jit, static_argnums=(2,))
def _emb_call(x_blocks, table, chunks):
    d = table.shape[1]
    rounds = chunks // _NBUF
    mesh = plsc.VectorSubcoreMesh(core_axis_name="c", subcore_axis_name="s")

    @functools.partial(
        pl.kernel,
        mesh=mesh,
        out_type=jax.ShapeDtypeStruct((_NW * chunks, _K, _CH, d), jnp.float32),
        scratch_types=[
            pltpu.VMEM((chunks, _K, _CH), jnp.int32),
            pltpu.VMEM((_NBUF, _K, _CH, d), jnp.float32),
            [pltpu.SemaphoreType.DMA] * _NBUF,
            [pltpu.SemaphoreType.DMA] * _NBUF,
        ],
        compiler_params=pltpu.CompilerParams(use_tc_tiling_on_sc=False),
    )
    def emb(x_hbm, table_hbm, out_hbm, idx_all, rows, gsems, osems):
        wid = lax.axis_index("s") * _NC + lax.axis_index("c")
        base = wid * chunks
        pltpu.sync_copy(x_hbm.at[wid], idx_all)

        def fire_gather(c, s):
            for b in range(_K):
                pltpu.async_copy(
                    table_hbm.at[idx_all.at[c, b]], rows.at[s, b], gsems[s])

        def drain_gather(c, s):
            for b in range(_K):
                pltpu.make_async_copy(
                    table_hbm.at[idx_all.at[c, b]], rows.at[s, b],
                    gsems[s]).wait()

        def fire_out(c, s):
            pltpu.async_copy(rows.at[s], out_hbm.at[base + c], osems[s])

        def wait_out(c, s):
            pltpu.make_async_copy(
                rows.at[s], out_hbm.at[base + c], osems[s]).wait()

        for s in range(_NBUF):
            fire_gather(s, s)

        def body(i, carry):
            for s in range(_NBUF):
                c = i * _NBUF + s
                drain_gather(c, s)
                fire_out(c, s)
            for s in range(_NBUF):
                c = i * _NBUF + s
                wait_out(c, s)
                fire_gather(c + _NBUF, s)
            return carry

        lax.fori_loop(0, rounds - 1, body, 0)

        for s in range(_NBUF):
            c = (rounds - 1) * _NBUF + s
            drain_gather(c, s)
            fire_out(c, s)
        for s in range(_NBUF):
            c = (rounds - 1) * _NBUF + s
            wait_out(c, s)

    return emb(x_blocks, table)


def kernel(x, table):
    b, l = x.shape
    _, d = table.shape
    n = b * l
    chunks = n // (_NW * _K * _CH)
    x_blocks = x.reshape(_NW, chunks, _K, _CH).astype(jnp.int32)
    out = _emb_call(x_blocks, table, chunks)
    return out.reshape(b, l, d)


# speedup vs baseline: 1.0161x; 1.0003x over previous
import functools

import jax
import jax.numpy as jnp
from jax import lax
from jax.experimental import pallas as pl
from jax.experimental.pallas import tpu as pltpu
from jax.experimental.pallas import tpu_sc as plsc

_NC = 2
_NS = 16
_NW = _NC * _NS

_NBUF = 4
_SA = 128


def _emb_call(x, table):
    b, l = x.shape
    d = table.shape[1]
    rpw = b // _NW
    rounds = rpw // _NBUF
    sb = l - _SA
    mesh = plsc.VectorSubcoreMesh(core_axis_name="c", subcore_axis_name="s")

    @functools.partial(
        pl.kernel,
        mesh=mesh,
        out_type=jax.ShapeDtypeStruct((b, l, d), jnp.float32),
        scratch_types=[
            pltpu.VMEM((rpw, l), jnp.int32),
            pltpu.VMEM((_NBUF, l, d), jnp.float32),
            [pltpu.SemaphoreType.DMA] * _NBUF,
            [pltpu.SemaphoreType.DMA] * _NBUF,
        ],
        compiler_params=pltpu.CompilerParams(use_tc_tiling_on_sc=False),
    )
    def emb(x_hbm, table_hbm, out_hbm, idx_all, rows, gsems, osems):
        wid = lax.axis_index("s") * _NC + lax.axis_index("c")
        base = wid * rpw
        pltpu.sync_copy(x_hbm.at[pl.ds(base, rpw)], idx_all)

        def fire_gather(r, s):
            pltpu.async_copy(
                table_hbm.at[idx_all.at[r, pl.ds(0, _SA)]],
                rows.at[s, pl.ds(0, _SA)], gsems[s])
            pltpu.async_copy(
                table_hbm.at[idx_all.at[r, pl.ds(_SA, sb)]],
                rows.at[s, pl.ds(_SA, sb)], gsems[s])

        def drain_gather(r, s):
            pltpu.make_async_copy(
                table_hbm.at[idx_all.at[r, pl.ds(0, _SA)]],
                rows.at[s, pl.ds(0, _SA)], gsems[s]).wait()
            pltpu.make_async_copy(
                table_hbm.at[idx_all.at[r, pl.ds(_SA, sb)]],
                rows.at[s, pl.ds(_SA, sb)], gsems[s]).wait()

        def fire_out(r, s):
            pltpu.async_copy(rows.at[s], out_hbm.at[base + r], osems[s])

        def wait_out(r, s):
            pltpu.make_async_copy(
                rows.at[s], out_hbm.at[base + r], osems[s]).wait()

        for s in range(_NBUF):
            fire_gather(s, s)

        def body(i, carry):
            for s in range(_NBUF):
                r = i * _NBUF + s
                drain_gather(r, s)
                fire_out(r, s)
            for s in range(_NBUF):
                r = i * _NBUF + s
                wait_out(r, s)
                fire_gather(r + _NBUF, s)
            return carry

        lax.fori_loop(0, rounds - 1, body, 0)

        for s in range(_NBUF):
            r = (rounds - 1) * _NBUF + s
            drain_gather(r, s)
            fire_out(r, s)
        for s in range(_NBUF):
            r = (rounds - 1) * _NBUF + s
            wait_out(r, s)

    return emb(x, table)


def kernel(x, table):
    return _emb_call(x.astype(jnp.int32), table)


# speedup vs baseline: 1.2350x; 1.2155x over previous
import functools

import jax
import jax.numpy as jnp
from jax import lax
from jax.experimental import pallas as pl
from jax.experimental.pallas import tpu as pltpu
from jax.experimental.pallas import tpu_sc as plsc

_NC = 2
_NS = 16
_NW = _NC * _NS

_NBUF = 4
_CH = 128


def _emb_call(xf, tp, b, l):
    n = xf.shape[0]
    dp = tp.shape[1]
    d = dp // 2
    rpw = n // _NW
    chunks = rpw // _CH
    rounds = chunks // _NBUF
    mesh = plsc.VectorSubcoreMesh(core_axis_name="c", subcore_axis_name="s")

    @functools.partial(
        pl.kernel,
        mesh=mesh,
        out_type=jax.ShapeDtypeStruct((n, dp), jnp.float32),
        scratch_types=[
            pltpu.VMEM((rpw,), jnp.int32),
            pltpu.VMEM((_NBUF, _CH, dp), jnp.float32),
            [pltpu.SemaphoreType.DMA] * _NBUF,
            [pltpu.SemaphoreType.DMA] * _NBUF,
        ],
        compiler_params=pltpu.CompilerParams(use_tc_tiling_on_sc=True),
    )
    def emb(x_hbm, table_hbm, out_hbm, idx_all, rows, gsems, osems):
        wid = lax.axis_index("s") * _NC + lax.axis_index("c")
        base = wid * rpw
        pltpu.sync_copy(x_hbm.at[pl.ds(base, rpw)], idx_all)

        def fire_gather(c, s):
            pltpu.async_copy(
                table_hbm.at[idx_all.at[pl.ds(c * _CH, _CH)]],
                rows.at[s], gsems[s])

        def drain_gather(c, s):
            pltpu.make_async_copy(
                table_hbm.at[idx_all.at[pl.ds(c * _CH, _CH)]],
                rows.at[s], gsems[s]).wait()

        def fire_out(c, s):
            pltpu.async_copy(
                rows.at[s], out_hbm.at[pl.ds(base + c * _CH, _CH)], osems[s])

        def wait_out(c, s):
            pltpu.make_async_copy(
                rows.at[s], out_hbm.at[pl.ds(base + c * _CH, _CH)],
                osems[s]).wait()

        for s in range(_NBUF):
            fire_gather(s, s)

        def body(i, carry):
            for s in range(_NBUF):
                c = i * _NBUF + s
                drain_gather(c, s)
                fire_out(c, s)
            for s in range(_NBUF):
                c = i * _NBUF + s
                wait_out(c, s)
                fire_gather(c + _NBUF, s)
            return carry

        lax.fori_loop(0, rounds - 1, body, 0)

        for s in range(_NBUF):
            c = (rounds - 1) * _NBUF + s
            drain_gather(c, s)
            fire_out(c, s)
        for s in range(_NBUF):
            c = (rounds - 1) * _NBUF + s
            wait_out(c, s)

    return emb(xf, tp)


def kernel(x, table):
    b, l = x.shape
    v, d = table.shape
    xf = x.reshape(b * l).astype(jnp.int32)
    tp = jnp.pad(table, ((0, 0), (0, d)))
    out128 = _emb_call(xf, tp, b, l)
    return out128[:, :d].reshape(b, l, d)


# speedup vs baseline: 1.2350x; 1.0000x over previous
import functools

import jax
import jax.numpy as jnp
from jax import lax
from jax.experimental import pallas as pl
from jax.experimental.pallas import tpu as pltpu
from jax.experimental.pallas import tpu_sc as plsc

_NC = 2
_NS = 16
_NW = _NC * _NS

_NBUF = 5
_CH = 128


def _emb_call(xf, tp):
    n = xf.shape[0]
    dp = tp.shape[1]
    rpw = n // _NW
    chunks = rpw // _CH
    rounds = chunks // _NBUF
    mesh = plsc.VectorSubcoreMesh(core_axis_name="c", subcore_axis_name="s")

    @functools.partial(
        pl.kernel,
        mesh=mesh,
        out_type=jax.ShapeDtypeStruct((n, dp), jnp.float32),
        scratch_types=[
            pltpu.VMEM((rpw,), jnp.int32),
            pltpu.VMEM((_NBUF, _CH, dp), jnp.float32),
            [pltpu.SemaphoreType.DMA] * _NBUF,
            [pltpu.SemaphoreType.DMA] * _NBUF,
        ],
        compiler_params=pltpu.CompilerParams(use_tc_tiling_on_sc=True),
    )
    def emb(x_hbm, table_hbm, out_hbm, idx_all, rows, gsems, osems):
        wid = lax.axis_index("s") * _NC + lax.axis_index("c")
        base = wid * rpw
        pltpu.sync_copy(x_hbm.at[pl.ds(base, rpw)], idx_all)

        def fire_gather(c, s):
            pltpu.async_copy(
                table_hbm.at[idx_all.at[pl.ds(c * _CH, _CH)]],
                rows.at[s], gsems[s])

        def drain_gather(c, s):
            pltpu.make_async_copy(
                table_hbm.at[idx_all.at[pl.ds(c * _CH, _CH)]],
                rows.at[s], gsems[s]).wait()

        def fire_out(c, s):
            pltpu.async_copy(
                rows.at[s], out_hbm.at[pl.ds(base + c * _CH, _CH)], osems[s])

        def wait_out(c, s):
            pltpu.make_async_copy(
                rows.at[s], out_hbm.at[pl.ds(base + c * _CH, _CH)],
                osems[s]).wait()

        for s in range(_NBUF):
            fire_gather(s, s)

        def body(i, carry):
            for s in range(_NBUF):
                c = i * _NBUF + s
                drain_gather(c, s)
                fire_out(c, s)
            for s in range(_NBUF):
                c = i * _NBUF + s
                wait_out(c, s)
                fire_gather(c + _NBUF, s)
            return carry

        lax.fori_loop(0, rounds - 1, body, 0)

        for s in range(_NBUF):
            c = (rounds - 1) * _NBUF + s
            drain_gather(c, s)
            fire_out(c, s)
        for s in range(_NBUF):
            c = (rounds - 1) * _NBUF + s
            wait_out(c, s)

    return emb(xf, tp)


def kernel(x, table):
    b, l = x.shape
    _, d = table.shape
    xf = x.reshape(b * l).astype(jnp.int32)
    tp = jnp.pad(table, ((0, 0), (0, d)))
    out128 = _emb_call(xf, tp)
    return out128[:, :d].reshape(b, l, d)


# speedup vs baseline: 1.2374x; 1.0019x over previous
import functools

import jax
import jax.numpy as jnp
from jax import lax
from jax.experimental import pallas as pl
from jax.experimental.pallas import tpu as pltpu
from jax.experimental.pallas import tpu_sc as plsc

_NC = 2
_NS = 16
_NW = _NC * _NS

_NBUF = 4
_CH = 128


def _emb_call(xf, tp):
    n = xf.shape[0]
    dp = tp.shape[1]
    rpw = n // _NW
    chunks = rpw // _CH
    rounds = chunks // _NBUF
    mesh = plsc.VectorSubcoreMesh(core_axis_name="c", subcore_axis_name="s")

    @functools.partial(
        pl.kernel,
        mesh=mesh,
        out_type=jax.ShapeDtypeStruct((n, dp), jnp.float32),
        scratch_types=[
            pltpu.VMEM((rpw,), jnp.int32),
            pltpu.VMEM((_NBUF, _CH, dp), jnp.float32),
            [pltpu.SemaphoreType.DMA] * _NBUF,
            [pltpu.SemaphoreType.DMA] * _NBUF,
        ],
        compiler_params=pltpu.CompilerParams(use_tc_tiling_on_sc=True),
    )
    def emb(x_hbm, table_hbm, out_hbm, idx_all, rows, gsems, osems):
        wid = lax.axis_index("s") * _NC + lax.axis_index("c")
        base = wid * rpw
        pltpu.sync_copy(x_hbm.at[pl.ds(base, rpw)], idx_all)

        def fire_gather(c, s):
            pltpu.async_copy(
                table_hbm.at[idx_all.at[pl.ds(c * _CH, _CH)]],
                rows.at[s], gsems[s])

        def drain_gather(c, s):
            pltpu.make_async_copy(
                table_hbm.at[idx_all.at[pl.ds(c * _CH, _CH)]],
                rows.at[s], gsems[s]).wait()

        def fire_out(c, s):
            pltpu.async_copy(
                rows.at[s], out_hbm.at[pl.ds(base + c * _CH, _CH)], osems[s])

        def wait_out(c, s):
            pltpu.make_async_copy(
                rows.at[s], out_hbm.at[pl.ds(base + c * _CH, _CH)],
                osems[s]).wait()

        for s in range(_NBUF):
            fire_gather(s, s)

        def body(i, carry):
            for s in range(_NBUF):
                c = i * _NBUF + s
                drain_gather(c, s)
                fire_out(c, s)
            for s in range(_NBUF):
                c = i * _NBUF + s
                wait_out(c, s)
                fire_gather(c + _NBUF, s)
            return carry

        lax.fori_loop(0, rounds - 1, body, 0)

        for s in range(_NBUF):
            c = (rounds - 1) * _NBUF + s
            drain_gather(c, s)
            fire_out(c, s)
        for s in range(_NBUF):
            c = (rounds - 1) * _NBUF + s
            wait_out(c, s)

    return emb(xf, tp)


def kernel(x, table):
    b, l = x.shape
    _, d = table.shape
    xf = x.reshape(b * l).astype(jnp.int32)
    tp = jnp.pad(table, ((0, 0), (0, d)))
    out128 = _emb_call(xf, tp)
    return out128[:, :d].reshape(b, l, d)
